# Initial kernel scaffold; baseline (speedup 1.0000x reference)
import jax
import jax.numpy as jnp
from jax.experimental import pallas as pl


def kernel(user_emb, item_emb, adj_values, adj_indices):
    raise NotImplementedError("write your pallas kernel here")


# speedup vs baseline: 4.4852x; 4.4852x over previous
import functools

import jax
import jax.numpy as jnp
from jax import lax
from jax.experimental import pallas as pl
from jax.experimental.pallas import tpu as pltpu
from jax.experimental.pallas import tpu_sc as plsc

N_USERS = 5000
N_NODES = 10000
EMB = 128
N_EDGES = 320000

NC = 2
NS = 16
DC = EMB // NC
NG = DC // 16
N_PAD = 10240
CHUNK = 64
BATCH = 8
RPT = 320
NB = RPT // BATCH
ROWS = RPT * NS
E_PAD = ROWS * CHUNK
EGO_R = N_PAD // NS
ZR = 32
OUT_ROWS = 5120
ACC_R = OUT_ROWS // NS
ACC_N = OUT_ROWS + 128
NK = ACC_R // ZR


def _b16(x):
    return jnp.full((16,), x, dtype=jnp.int32)


_GDN = lax.GatherDimensionNumbers(
    offset_dims=(), collapsed_slice_dims=(0,), start_index_map=(0,))


def _bcast_lane(vv, l):
    return lax.gather(vv, _b16(l)[:, None], dimension_numbers=_GDN,
                      slice_sizes=(1,),
                      mode=lax.GatherScatterMode.PROMISE_IN_BOUNDS)


_mesh = plsc.VectorSubcoreMesh(core_axis_name="c", subcore_axis_name="s")


@functools.partial(
    pl.kernel,
    out_type=jax.ShapeDtypeStruct((NC, OUT_ROWS, DC), jnp.float32),
    mesh=_mesh,
    compiler_params=pltpu.CompilerParams(use_tc_tiling_on_sc=False),
    scratch_types=[
        pltpu.VMEM_SHARED((N_PAD, DC), jnp.float32),
        pltpu.VMEM_SHARED((N_PAD, DC), jnp.float32),
        pltpu.VMEM_SHARED((ACC_N, DC), jnp.float32),
        pltpu.VMEM((BATCH, CHUNK), jnp.int32),
        pltpu.VMEM((BATCH, CHUNK), jnp.int32),
        pltpu.VMEM((BATCH, CHUNK), jnp.int32),
        pltpu.VMEM((BATCH * CHUNK,), jnp.float32),
        pltpu.VMEM((CHUNK, DC), jnp.float32),
        pltpu.VMEM((ZR, DC), jnp.float32),
        pltpu.VMEM((ZR, DC), jnp.float32),
        pltpu.VMEM((ACC_R // 16, 16), jnp.int32),
    ],
)
def _gnn_sc(ego_hbm, col_hbm, row_hbm, val_hbm, out_hbm,
            tab_a, tab_b, acc, col_b, row_b, crow_b, val_b, msg,
            zbuf, f1, idx_acc):
    c = lax.axis_index("c")
    s = lax.axis_index("s")
    r0 = pl.multiple_of(s * RPT, RPT)
    e0 = pl.multiple_of(s * EGO_R, EGO_R)
    a0 = pl.multiple_of(s * ACC_R, ACC_R)

    zv = jnp.zeros((16,), jnp.float32)

    def zb_body(r, carry):
        for g in range(NG):
            zbuf[r, pl.ds(g * 16, 16)] = zv
        return carry

    lax.fori_loop(0, ZR, zb_body, 0)

    lanes = lax.iota(jnp.int32, 16)
    for k in range(ACC_R // 16):
        idx_acc[k, :] = lanes + _b16(a0 + k * 16)

    def init_body(k, carry):
        rr = pl.multiple_of(e0 + k * ZR, ZR)
        pltpu.sync_copy(ego_hbm.at[c, pl.ds(rr, ZR), :], f1)
        pltpu.sync_copy(f1, tab_a.at[pl.ds(rr, ZR), :])
        pltpu.sync_copy(zbuf, tab_b.at[pl.ds(rr, ZR), :])
        return carry

    lax.fori_loop(0, EGO_R // ZR, init_body, 0)
    plsc.subcore_barrier()

    def edge_pass(cur, dst, clamp):
        def batch_body(jj, carry):
            rr = pl.multiple_of(r0 + jj * BATCH, BATCH)
            pltpu.sync_copy(col_hbm.at[pl.ds(rr, BATCH), :], col_b)
            pltpu.sync_copy(row_hbm.at[pl.ds(rr, BATCH), :], row_b)
            pltpu.sync_copy(val_hbm.at[pl.ds(rr * CHUNK, BATCH * CHUNK)],
                            val_b)

            for j8 in range(BATCH):
                pltpu.sync_copy(cur.at[col_b.at[j8]], msg)

                def grp_body(g, carry3, j8=j8):
                    off = pl.multiple_of(j8 * CHUNK + g * 16, 16)
                    vv = val_b[pl.ds(off, 16)]
                    for l in range(16):
                        bv = _bcast_lane(vv, l)
                        e = g * 16 + l
                        for q in range(NG):
                            sl = pl.ds(q * 16, 16)
                            msg[e, sl] = msg[e, sl] * bv
                    return carry3

                lax.fori_loop(0, CHUNK // 16, grp_body, 0)
                if clamp:
                    cmax = _b16(OUT_ROWS)
                    for g in range(CHUNK // 16):
                        sl = pl.ds(g * 16, 16)
                        crow_b[j8, sl] = jnp.minimum(row_b[j8, sl], cmax)
                    pltpu.sync_copy(msg, dst.at[crow_b.at[j8]], add=True)
                else:
                    pltpu.sync_copy(msg, dst.at[row_b.at[j8]], add=True)
            return carry

        lax.fori_loop(0, NB, batch_body, 0)

    edge_pass(tab_a, tab_b, clamp=False)
    plsc.subcore_barrier()

    def acccopy_body(k, carry):
        rr = pl.multiple_of(a0 + k * ZR, ZR)
        pltpu.sync_copy(tab_b.at[pl.ds(rr, ZR), :], f1)
        pltpu.sync_copy(f1, acc.at[pl.ds(rr, ZR), :])
        return carry

    lax.fori_loop(0, NK, acccopy_body, 0)

    def zero_a_body(k, carry):
        rr = pl.multiple_of(e0 + k * ZR, ZR)
        pltpu.sync_copy(zbuf, tab_a.at[pl.ds(rr, ZR), :])
        return carry

    lax.fori_loop(0, EGO_R // ZR, zero_a_body, 0)
    plsc.subcore_barrier()

    edge_pass(tab_b, tab_a, clamp=False)
    plsc.subcore_barrier()

    for k in range(NK):
        rr = pl.multiple_of(a0 + k * ZR, ZR)
        pltpu.sync_copy(tab_a.at[pl.ds(rr, ZR), :], f1)
        for h in range(ZR // 16):
            pltpu.sync_copy(f1.at[pl.ds(h * 16, 16), :],
                            acc.at[idx_acc.at[k * (ZR // 16) + h]],
                            add=True)

    edge_pass(tab_a, acc, clamp=True)
    plsc.subcore_barrier()

    third = jnp.full((16,), 1.0 / 3.0, dtype=jnp.float32)

    def out_body(k, carry):
        rr = pl.multiple_of(a0 + k * ZR, ZR)
        pltpu.sync_copy(acc.at[pl.ds(rr, ZR), :], f1)

        def mul_body(r, carry2):
            for g in range(NG):
                sl = pl.ds(g * 16, 16)
                f1[r, sl] = f1[r, sl] * third
            return carry2

        lax.fori_loop(0, ZR, mul_body, 0)
        pltpu.sync_copy(f1, out_hbm.at[c, pl.ds(rr, ZR), :])
        return carry

    lax.fori_loop(0, NK, out_body, 0)


def kernel(user_emb, item_emb, adj_values, adj_indices):
    ego0 = jnp.concatenate(
        [user_emb, item_emb,
         jnp.zeros((N_PAD - N_NODES, EMB), jnp.float32)], axis=0)
    ego0 = ego0.reshape(N_PAD, NC, DC).transpose(1, 0, 2)
    idx = adj_indices.astype(jnp.int32)
    pad = E_PAD - N_EDGES
    row = jnp.concatenate(
        [idx[0], jnp.full((pad,), N_PAD - 1, jnp.int32)]).reshape(ROWS, CHUNK)
    col = jnp.concatenate(
        [idx[1], jnp.zeros((pad,), jnp.int32)]).reshape(ROWS, CHUNK)
    val = jnp.concatenate([adj_values, jnp.zeros((pad,), jnp.float32)])
    out = _gnn_sc(ego0, col, row, val)
    user_all = out.transpose(1, 0, 2).reshape(OUT_ROWS, EMB)[:N_USERS]
    return user_all, item_emb
